# Initial kernel scaffold; baseline (speedup 1.0000x reference)
import jax
import jax.numpy as jnp
from jax.experimental import pallas as pl


def kernel(x, edge_index_stock, edge_index_supplies_to, edge_index_supplies_from, params):
    raise NotImplementedError("write your pallas kernel here")


# speedup vs baseline: 65.9556x; 65.9556x over previous
import functools

import jax
import jax.numpy as jnp
from jax import lax
from jax.experimental import pallas as pl
from jax.experimental.pallas import tpu as pltpu
from jax.experimental.pallas import tpu_sc as plsc

N = 5000
T = 20
F = 128
H = 16
FT = T * H
E = 80000

NC = 2
NS = 16
NW = NC * NS
CHUNK = 32
CPT = 80
EP = NW * CPT * CHUNK
ACC_ROWS = 5120
RPT = ACC_ROWS // NS

NB = 1000
NGB = N // NB

_sc_mesh = plsc.VectorSubcoreMesh(core_axis_name="c", subcore_axis_name="s",
                                  num_cores=NC, num_subcores=NS)


DEG_W = 16
DEG_ROWS = 3 * ACC_ROWS
DEG_RPT = DEG_ROWS // NS


@functools.partial(
    pl.kernel,
    out_type=jax.ShapeDtypeStruct((NC, DEG_ROWS, DEG_W), jnp.float32),
    mesh=_sc_mesh,
    compiler_params=pltpu.CompilerParams(use_tc_tiling_on_sc=False),
    scratch_types=[
        pltpu.VMEM((3 * CPT, CHUNK), jnp.int32),
        pltpu.VMEM((CHUNK, DEG_W), jnp.float32),
        pltpu.VMEM((DEG_RPT, DEG_W), jnp.float32),
        pltpu.VMEM_SHARED((DEG_ROWS, DEG_W), jnp.float32),
    ],
)
def _deg_kernel(dst3_hbm, out_hbm, dst_v, ones_v, zero_v, acc):
    c = lax.axis_index("c")
    s = lax.axis_index("s")
    wid = s * NC + c
    pltpu.sync_copy(dst3_hbm.at[wid], dst_v)

    def fill(i, _):
        ones_v[i, :] = jnp.ones((DEG_W,), jnp.float32)
        zero_v[i, :] = jnp.zeros((DEG_W,), jnp.float32)
        return 0

    lax.fori_loop(0, CHUNK, fill, 0)

    def fillz(i, _):
        zero_v[i, :] = jnp.zeros((DEG_W,), jnp.float32)
        return 0

    lax.fori_loop(CHUNK, DEG_RPT, fillz, 0)
    pltpu.sync_copy(zero_v, acc.at[pl.ds(s * DEG_RPT, DEG_RPT)])
    plsc.subcore_barrier()

    def body(k, _):
        pltpu.sync_copy(ones_v, acc.at[dst_v.at[k]], add=True)
        return 0

    lax.fori_loop(0, 3 * CPT, body, 0)
    plsc.subcore_barrier()
    pltpu.sync_copy(acc.at[pl.ds(s * DEG_RPT, DEG_RPT)],
                    out_hbm.at[c, pl.ds(s * DEG_RPT, DEG_RPT)])


@functools.partial(
    pl.kernel,
    out_type=jax.ShapeDtypeStruct((NC, ACC_ROWS, FT), jnp.float32),
    mesh=_sc_mesh,
    compiler_params=pltpu.CompilerParams(use_tc_tiling_on_sc=False),
    scratch_types=[
        pltpu.VMEM((CPT, CHUNK), jnp.int32),
        pltpu.VMEM((CPT, CHUNK), jnp.int32),
        pltpu.VMEM((CHUNK, FT), jnp.float32),
        pltpu.VMEM((CHUNK, FT), jnp.float32),
        pltpu.VMEM_SHARED((ACC_ROWS, FT), jnp.float32),
        pltpu.SemaphoreType.DMA,
        pltpu.SemaphoreType.DMA,
    ],
)
def _conv_kernel(xw_hbm, src_hbm, dst_hbm, out_hbm,
                 src_v, dst_v, rows_a, rows_b, acc, sem_a, sem_b):
    c = lax.axis_index("c")
    s = lax.axis_index("s")
    wid = s * NC + c
    pltpu.sync_copy(src_hbm.at[wid], src_v)
    pltpu.sync_copy(dst_hbm.at[wid], dst_v)

    def fill(i, _):
        for j in range(FT // 16):
            rows_a[i, pl.ds(j * 16, 16)] = jnp.zeros((16,), jnp.float32)
        return 0

    lax.fori_loop(0, CHUNK, fill, 0)
    base = s * RPT
    for m in range(RPT // CHUNK):
        pltpu.sync_copy(rows_a, acc.at[pl.ds(base + m * CHUNK, CHUNK)])
    plsc.subcore_barrier()

    def body(k, _):
        d0 = pltpu.async_copy(xw_hbm.at[src_v.at[2 * k]], rows_a, sem_a)
        d1 = pltpu.async_copy(xw_hbm.at[src_v.at[2 * k + 1]], rows_b, sem_b)
        d0.wait()
        pltpu.sync_copy(rows_a, acc.at[dst_v.at[2 * k]], add=True)
        d1.wait()
        pltpu.sync_copy(rows_b, acc.at[dst_v.at[2 * k + 1]], add=True)
        return 0

    lax.fori_loop(0, CPT // 2, body, 0)
    plsc.subcore_barrier()
    pltpu.sync_copy(acc.at[pl.ds(s * RPT, RPT)],
                    out_hbm.at[c, pl.ds(s * RPT, RPT)])


def _leaky(v):
    return jnp.where(v >= 0, v, 0.01 * v)


def _lstm_body(x_ref, wih0_ref, whh0_ref, b0_ref, wih1_ref, whh1_ref, b1_ref,
               nf_ref):
    h0 = jnp.zeros((NB, H), jnp.float32)
    c0 = jnp.zeros((NB, H), jnp.float32)
    h1 = jnp.zeros((NB, H), jnp.float32)
    c1 = jnp.zeros((NB, H), jnp.float32)
    wih0 = wih0_ref[...]
    whh0 = whh0_ref[...]
    b0 = b0_ref[...]
    wih1 = wih1_ref[...]
    whh1 = whh1_ref[...]
    b1 = b1_ref[...]
    outs = []
    for t in range(T):
        xt = x_ref[:, t, :]
        g0 = (jnp.dot(xt, wih0, preferred_element_type=jnp.float32)
              + jnp.dot(h0, whh0, preferred_element_type=jnp.float32) + b0)
        i0 = jax.nn.sigmoid(g0[:, 0:H])
        f0 = jax.nn.sigmoid(g0[:, H:2 * H])
        gg0 = jnp.tanh(g0[:, 2 * H:3 * H])
        o0 = jax.nn.sigmoid(g0[:, 3 * H:4 * H])
        c0 = f0 * c0 + i0 * gg0
        h0 = o0 * jnp.tanh(c0)
        g1 = (jnp.dot(h0, wih1, preferred_element_type=jnp.float32)
              + jnp.dot(h1, whh1, preferred_element_type=jnp.float32) + b1)
        i1 = jax.nn.sigmoid(g1[:, 0:H])
        f1 = jax.nn.sigmoid(g1[:, H:2 * H])
        gg1 = jnp.tanh(g1[:, 2 * H:3 * H])
        o1 = jax.nn.sigmoid(g1[:, 3 * H:4 * H])
        c1 = f1 * c1 + i1 * gg1
        h1 = o1 * jnp.tanh(c1)
        outs.append(_leaky(h1))
    nf_ref[...] = jnp.concatenate(outs, axis=1)


def _lstm_tc(x, wih0t, whh0t, b0, wih1t, whh1t, b1):
    return pl.pallas_call(
        _lstm_body,
        grid=(NGB,),
        in_specs=[
            pl.BlockSpec((NB, T, F), lambda i: (i, 0, 0)),
            pl.BlockSpec((F, 4 * H), lambda i: (0, 0)),
            pl.BlockSpec((H, 4 * H), lambda i: (0, 0)),
            pl.BlockSpec((1, 4 * H), lambda i: (0, 0)),
            pl.BlockSpec((H, 4 * H), lambda i: (0, 0)),
            pl.BlockSpec((H, 4 * H), lambda i: (0, 0)),
            pl.BlockSpec((1, 4 * H), lambda i: (0, 0)),
        ],
        out_specs=pl.BlockSpec((NB, FT), lambda i: (i, 0)),
        out_shape=jax.ShapeDtypeStruct((N, FT), jnp.float32),
    )(x, wih0t, whh0t, b0, wih1t, whh1t, b1)


def _dis_body(deg_ref, out_ref):
    out_ref[...] = lax.rsqrt(deg_ref[0] + deg_ref[1] + 2.0)


def _dis_tc(deg_parts):
    return pl.pallas_call(
        _dis_body,
        out_shape=jax.ShapeDtypeStruct((DEG_ROWS, DEG_W), jnp.float32),
    )(deg_parts)


def _xw0_body(nf_ref, k_ref, dis_ref, out_ref):
    z = jnp.dot(nf_ref[...], k_ref[0],
                preferred_element_type=jnp.float32)
    out_ref[0] = z * dis_ref[0, :, :1]


def _xw0_tc(nf, kstack, dis3):
    return pl.pallas_call(
        _xw0_body,
        grid=(3, NGB),
        in_specs=[
            pl.BlockSpec((NB, FT), lambda r, i: (i, 0)),
            pl.BlockSpec((1, FT, FT), lambda r, i: (r, 0, 0)),
            pl.BlockSpec((1, NB, DEG_W), lambda r, i: (r, i, 0)),
        ],
        out_specs=pl.BlockSpec((1, NB, FT), lambda r, i: (r, i, 0)),
        out_shape=jax.ShapeDtypeStruct((3, N, FT), jnp.float32),
    )(nf, kstack, dis3)


def _combine_mid_body(p_ref, xw_ref, dis_ref, kn_ref, b_ref, out_ref):
    dis = dis_ref[:, :1]
    y = _leaky(dis * (p_ref[0] + p_ref[1] + 2.0 * xw_ref[...]) + b_ref[...])
    out_ref[...] = jnp.dot(y, kn_ref[...],
                           preferred_element_type=jnp.float32) * dis


def _combine_mid_tc(p, xw, dis, kn, btile):
    return pl.pallas_call(
        _combine_mid_body,
        grid=(NGB,),
        in_specs=[
            pl.BlockSpec((NC, NB, FT), lambda i: (0, i, 0)),
            pl.BlockSpec((NB, FT), lambda i: (i, 0)),
            pl.BlockSpec((NB, DEG_W), lambda i: (i, 0)),
            pl.BlockSpec((FT, FT), lambda i: (0, 0)),
            pl.BlockSpec((1, FT), lambda i: (0, 0)),
        ],
        out_specs=pl.BlockSpec((NB, FT), lambda i: (i, 0)),
        out_shape=jax.ShapeDtypeStruct((N, FT), jnp.float32),
    )(p, xw, dis, kn, btile)


def _combine_final_body(p_ref, xw_ref, dis_ref, b_ref, out_ref):
    dis = dis_ref[:, :1]
    out_ref[...] = _leaky(
        dis * (p_ref[0] + p_ref[1] + 2.0 * xw_ref[...]) + b_ref[...])


def _combine_final_tc(p, xw, dis, btile):
    return pl.pallas_call(
        _combine_final_body,
        grid=(NGB,),
        in_specs=[
            pl.BlockSpec((NC, NB, FT), lambda i: (0, i, 0)),
            pl.BlockSpec((NB, FT), lambda i: (i, 0)),
            pl.BlockSpec((NB, DEG_W), lambda i: (i, 0)),
            pl.BlockSpec((1, FT), lambda i: (0, 0)),
        ],
        out_specs=pl.BlockSpec((NB, FT), lambda i: (i, 0)),
        out_shape=jax.ShapeDtypeStruct((N, FT), jnp.float32),
    )(p, xw, dis, btile)


def _head_body(zs_ref, zt_ref, zf_ref, as_ref, at_ref, af_ref, b_ref, out_ref):
    out_ref[...] = (
        jnp.dot(zs_ref[...], as_ref[...], preferred_element_type=jnp.float32)
        + jnp.dot(zt_ref[...], at_ref[...], preferred_element_type=jnp.float32)
        + jnp.dot(zf_ref[...], af_ref[...], preferred_element_type=jnp.float32)
        + b_ref[...])


def _head_tc(zs, zt, zf, a_s, a_t, a_f, btot):
    return pl.pallas_call(
        _head_body,
        grid=(NGB,),
        in_specs=[
            pl.BlockSpec((NB, FT), lambda i: (i, 0)),
            pl.BlockSpec((NB, FT), lambda i: (i, 0)),
            pl.BlockSpec((NB, FT), lambda i: (i, 0)),
            pl.BlockSpec((FT, T), lambda i: (0, 0)),
            pl.BlockSpec((FT, T), lambda i: (0, 0)),
            pl.BlockSpec((FT, T), lambda i: (0, 0)),
            pl.BlockSpec((1, T), lambda i: (0, 0)),
        ],
        out_specs=pl.BlockSpec((NB, T), lambda i: (i, 0)),
        out_shape=jax.ShapeDtypeStruct((N, T), jnp.float32),
    )(zs, zt, zf, a_s, a_t, a_f, btot)


def _prep_edges(ei):
    src = jnp.concatenate(
        [ei[0], jnp.zeros((EP - E,), jnp.int32)]).reshape(NW, CPT, CHUNK)
    dst = jnp.concatenate(
        [ei[1], jnp.full((EP - E,), N, jnp.int32)]).reshape(NW, CPT, CHUNK)
    return src, dst


def _kron20(w):
    a, b = w.shape
    eye = jnp.eye(T, dtype=jnp.float32)
    return (eye[:, None, :, None] * w[None, :, None, :]).reshape(T * a, T * b)


def kernel(x, edge_index_stock, edge_index_supplies_to,
           edge_index_supplies_from, params):
    p = params
    eis = (edge_index_stock, edge_index_supplies_to, edge_index_supplies_from)
    rels = ('stock', 'to', 'from')

    packed = [_prep_edges(ei) for ei in eis]
    dst3 = jnp.stack([packed[r][1].reshape(NW, CPT, CHUNK) + r * ACC_ROWS
                      for r in range(3)], axis=1).reshape(NW, 3 * CPT, CHUNK)

    b0 = (p['bih0'] + p['bhh0'])[None, :]
    b1 = (p['bih1'] + p['bhh1'])[None, :]
    kw0 = jnp.stack([_kron20(p[f'{r}_W0']) for r in rels])
    kw1 = [_kron20(p[f'{r}_W1']) for r in rels]
    bt0 = [jnp.tile(p[f'{r}_b0'], T)[None, :] for r in rels]
    bt1 = [jnp.tile(p[f'{r}_b1'], T)[None, :] for r in rels]
    w1k = _kron20(p['lin_W1'])
    w2k = _kron20(p['lin_W2'])
    w3k = _kron20(p['lin_W3'])
    a_rel = [(_kron20(p['lin_W0'][r * H:(r + 1) * H, :]) @ w1k @ w2k @ w3k)
             for r in range(3)]
    btot = ((jnp.tile(p['lin_b0'], T) @ w1k + jnp.tile(p['lin_b1'], T))
            @ w2k + jnp.tile(p['lin_b2'], T)) @ w3k + jnp.tile(p['lin_b3'], T)
    btot = btot[None, :]

    deg_parts = _deg_kernel(dst3)
    nf = _lstm_tc(x, p['Wih0'].T, p['Whh0'].T, b0, p['Wih1'].T, p['Whh1'].T,
                  b1)
    dis_full = _dis_tc(deg_parts).reshape(3, ACC_ROWS, DEG_W)

    xw0 = _xw0_tc(nf, kw0, dis_full[:, :N, :])

    zs = []
    for r in range(3):
        src, dst = packed[r]
        dis_r = dis_full[r, :N, :]
        p1 = _conv_kernel(xw0[r], src, dst)
        xw1 = _combine_mid_tc(p1, xw0[r], dis_r, kw1[r], bt0[r])
        p2 = _conv_kernel(xw1, src, dst)
        zs.append(_combine_final_tc(p2, xw1, dis_r, bt1[r]))

    return _head_tc(zs[0], zs[1], zs[2], a_rel[0], a_rel[1], a_rel[2], btot)
